# Initial kernel scaffold; baseline (speedup 1.0000x reference)
import jax
import jax.numpy as jnp
from jax.experimental import pallas as pl


def kernel(node_features, edge_features, from_idx, to_idx, enc_node_W, enc_node_b, enc_edge_W, enc_edge_b, msg_W1, msg_b1, msg_W2, msg_b2, upd_W, upd_b, fc1_W, fc1_b, fc2_W, fc2_b):
    raise NotImplementedError("write your pallas kernel here")


# speedup vs baseline: 1.0085x; 1.0085x over previous
import functools

import jax
import jax.numpy as jnp
from jax.experimental import pallas as pl
from jax.experimental.pallas import tpu as pltpu

N_PAIRS = 2048
N_GRAPHS = 2 * N_PAIRS
NODES_PER_GRAPH = 15
MAX_SET_SIZE = 20
N_NODES = N_GRAPHS * NODES_PER_GRAPH
EDGES_PER_GRAPH = 60
N_EDGES = N_GRAPHS * EDGES_PER_GRAPH
NODE_STATE_DIM = 64
N_PROP_LAYERS = 5
SINKHORN_TEMP = 0.1
SINKHORN_ITERS = 20


def _enc_body(nf_ref, w_ref, b_ref, out_ref):
    out_ref[...] = jnp.dot(nf_ref[...], w_ref[...],
                           preferred_element_type=jnp.float32) + b_ref[...]


def _encode_nodes(node_features, W, b):
    blk = 7680
    grid = N_NODES // blk
    return pl.pallas_call(
        _enc_body,
        grid=(grid,),
        in_specs=[
            pl.BlockSpec((blk, node_features.shape[1]), lambda i: (i, 0)),
            pl.BlockSpec(W.shape, lambda i: (0, 0)),
            pl.BlockSpec((1, NODE_STATE_DIM), lambda i: (0, 0)),
        ],
        out_specs=pl.BlockSpec((blk, NODE_STATE_DIM), lambda i: (i, 0)),
        out_shape=jax.ShapeDtypeStruct((N_NODES, NODE_STATE_DIM), jnp.float32),
    )(node_features, W, b.reshape(1, -1))


def kernel(node_features, edge_features, from_idx, to_idx, enc_node_W,
           enc_node_b, enc_edge_W, enc_edge_b, msg_W1, msg_b1, msg_W2, msg_b2,
           upd_W, upd_b, fc1_W, fc1_b, fc2_W, fc2_b):
    h = _encode_nodes(node_features, enc_node_W, enc_node_b)
    e = edge_features @ enc_edge_W + enc_edge_b
    for _ in range(N_PROP_LAYERS):
        msg_in = jnp.concatenate([h[from_idx], h[to_idx], e], axis=1)
        msg = jax.nn.relu(msg_in @ msg_W1 + msg_b1) @ msg_W2 + msg_b2
        agg = jax.ops.segment_sum(msg, to_idx, num_segments=N_NODES)
        h = jnp.concatenate([h, agg], axis=1) @ upd_W + upd_b
    h = h.reshape(N_GRAPHS, NODES_PER_GRAPH, NODE_STATE_DIM)
    pad = jnp.zeros((N_GRAPHS, MAX_SET_SIZE - NODES_PER_GRAPH, NODE_STATE_DIM),
                    jnp.float32)
    h = jnp.concatenate([h, pad], axis=1)
    stacked_q = h[0::2]
    stacked_c = h[1::2]
    tq = jax.nn.relu(stacked_q @ fc1_W + fc1_b) @ fc2_W + fc2_b
    tc = jax.nn.relu(stacked_c @ fc1_W + fc1_b) @ fc2_W + fc2_b
    mask = (jnp.arange(MAX_SET_SIZE) < NODES_PER_GRAPH).astype(jnp.float32)[None, :, None]
    mq = tq * mask
    mc = tc * mask
    log_alpha = jnp.matmul(mq, jnp.transpose(mc, (0, 2, 1))) / SINKHORN_TEMP
    for _ in range(SINKHORN_ITERS):
        log_alpha = log_alpha - jax.nn.logsumexp(log_alpha, axis=2, keepdims=True)
        log_alpha = log_alpha - jax.nn.logsumexp(log_alpha, axis=1, keepdims=True)
    transport_plan = jnp.exp(log_alpha)
    scores = -jnp.sum(jnp.maximum(stacked_q - jnp.matmul(transport_plan, stacked_c), 0.0),
                      axis=(1, 2))
    return scores


# speedup vs baseline: 3.0326x; 3.0071x over previous
import functools

import jax
import jax.numpy as jnp
from jax import lax
from jax.experimental import pallas as pl
from jax.experimental.pallas import tpu as pltpu
from jax.experimental.pallas import tpu_sc as plsc

N_PAIRS = 2048
N_GRAPHS = 2 * N_PAIRS
NPG = 15
MAX_SET = 20
N_NODES = N_GRAPHS * NPG
EPG = 60
N_EDGES = N_GRAPHS * EPG
D = 64
D2 = 128
N_LAYERS = 5
TEMP = 0.1
SINK_ITERS = 20

NW = 32
G_CHUNK = 4
CHUNKS = N_GRAPHS // (NW * G_CHUNK)
CN = G_CHUNK * NPG
CE = G_CHUNK * EPG
CEP = 256
JROWS = CEP // 128
IDX_ROWS = N_EDGES // 120


def _enc_body(nf_ref, ef_ref, wn_ref, bn_ref, w1ab_ref, wec_ref,
              bec_ref, h_ref, pq_ref, e1_ref):
    h = jnp.dot(nf_ref[...], wn_ref[...],
                preferred_element_type=jnp.float32) + bn_ref[...]
    h_ref[...] = h
    pq_ref[...] = jnp.dot(h, w1ab_ref[...], preferred_element_type=jnp.float32)
    e1 = jnp.dot(ef_ref[...], wec_ref[...],
                 preferred_element_type=jnp.float32) + bec_ref[...]
    e3 = e1.reshape(64, 120, D)
    e3 = jnp.concatenate([e3, jnp.zeros((64, 8, D), jnp.float32)], axis=1)
    e1_ref[...] = e3.reshape(64 * 128, D)


def _encoder(nf, ef, Wn, bn, W1ab, Wec, bec):
    grid = 32
    nblk = N_NODES // grid
    eblk = N_EDGES // grid
    return pl.pallas_call(
        _enc_body,
        grid=(grid,),
        in_specs=[
            pl.BlockSpec((nblk, 32), lambda i: (i, 0)),
            pl.BlockSpec((eblk, 4), lambda i: (i, 0)),
            pl.BlockSpec((32, D), lambda i: (0, 0)),
            pl.BlockSpec((1, D), lambda i: (0, 0)),
            pl.BlockSpec((D, D2), lambda i: (0, 0)),
            pl.BlockSpec((4, D), lambda i: (0, 0)),
            pl.BlockSpec((1, D), lambda i: (0, 0)),
        ],
        out_specs=[
            pl.BlockSpec((nblk, D), lambda i: (i, 0)),
            pl.BlockSpec((nblk, D2), lambda i: (i, 0)),
            pl.BlockSpec((64 * 128, D), lambda i: (i, 0)),
        ],
        out_shape=[
            jax.ShapeDtypeStruct((N_NODES, D), jnp.float32),
            jax.ShapeDtypeStruct((N_NODES, D2), jnp.float32),
            jax.ShapeDtypeStruct((IDX_ROWS * 128, D), jnp.float32),
        ],
    )(nf, ef, Wn, bn.reshape(1, -1), W1ab, Wec, bec.reshape(1, -1))


def _edge_body(pq_hbm, e1_hbm, fidx_hbm, tidx_hbm, s_hbm,
               fidx_v, tidx_v, tloc_v, gf_v, gt_v, e1_v, zeros_v, s_sh):
    sid = lax.axis_index("s")
    w = sid * 2 + lax.axis_index("c")
    zero16 = jnp.zeros((16,), jnp.float32)
    one16 = jnp.ones((16,), jnp.float32)

    def fill_const(i, _):
        for c8 in range(8):
            zeros_v[i, pl.ds(c8 * 16, 16)] = zero16
        return 0
    lax.fori_loop(0, 40, fill_const, 0, unroll=4)

    pltpu.sync_copy(fidx_hbm.at[pl.ds(w * 64, 64)], fidx_v)
    pltpu.sync_copy(tidx_hbm.at[pl.ds(w * 64, 64)], tidx_v)

    sbase = 120 * sid
    pairs = CHUNKS // 2

    def pair_body(p, _):
        node0 = 2 * CN * (w * pairs + p)
        for z in range(3):
            pltpu.sync_copy(zeros_v, s_sh.at[pl.ds(sbase + 40 * z, 40)])
        for half in range(2):
            jrow0 = 4 * p + JROWS * half
            e1row0 = CEP * (2 * (w * pairs + p) + half)

            def loc_body(i, _):
                j = jrow0 + i // 8
                k = i % 8
                sl = pl.ds(k * 16, 16)
                tloc_v[i // 8, sl] = tidx_v[j, sl] - node0 + sbase
                return 0
            lax.fori_loop(0, 8 * JROWS, loc_body, 0, unroll=4)

            for j in range(JROWS):
                pltpu.sync_copy(pq_hbm.at[fidx_v.at[jrow0 + j]],
                                gf_v.at[pl.ds(j * 128, 128)])
                pltpu.sync_copy(pq_hbm.at[tidx_v.at[jrow0 + j]],
                                gt_v.at[pl.ds(j * 128, 128)])

            for sub in range(2):
                pltpu.sync_copy(e1_hbm.at[pl.ds(e1row0 + 128 * sub, 128)],
                                e1_v)

                def relu_body(i, _):
                    g = 128 * sub + i
                    for c4 in range(4):
                        sl = pl.ds(c4 * 16, 16)
                        v = (gf_v[g, sl] + gt_v[g, pl.ds(64 + c4 * 16, 16)]
                             + e1_v[i, sl])
                        gf_v[g, sl] = jnp.maximum(v, 0.0)
                    gf_v[g, pl.ds(64, 16)] = one16
                    return 0
                lax.fori_loop(0, 128, relu_body, 0, unroll=2)

            for j in range(JROWS):
                pltpu.sync_copy(gf_v.at[pl.ds(j * 128, 128)],
                                s_sh.at[tloc_v.at[j]], add=True)

        pltpu.sync_copy(s_sh.at[pl.ds(sbase, 2 * CN)],
                        s_hbm.at[pl.ds(node0, 2 * CN)])
        return 0

    lax.fori_loop(0, pairs, pair_body, 0)


def _edge_stage(PQ, E1p, fr, tr):
    mesh = plsc.VectorSubcoreMesh(core_axis_name="c", subcore_axis_name="s")
    f = functools.partial(
        pl.kernel,
        out_type=jax.ShapeDtypeStruct((N_NODES, D2), jnp.float32),
        mesh=mesh,
        scratch_types=[
            pltpu.VMEM((64, 128), jnp.int32),
            pltpu.VMEM((64, 128), jnp.int32),
            pltpu.VMEM((JROWS, 128), jnp.int32),
            pltpu.VMEM((CEP, D2), jnp.float32),
            pltpu.VMEM((CEP, D2), jnp.float32),
            pltpu.VMEM((128, D), jnp.float32),
            pltpu.VMEM((40, D2), jnp.float32),
            pltpu.VMEM_SHARED((16 * 120 + 8, D2), jnp.float32),
        ],
    )(_edge_body)
    return f(PQ, E1p, fr, tr)


def _upd_body_full(h_ref, s_ref, wua_ref, w2c_ref, b16_ref, ub_ref,
                   w1ab_ref, h2_ref, pq_ref):
    s = s_ref[...]
    h2 = (jnp.dot(h_ref[...], wua_ref[...], preferred_element_type=jnp.float32)
          + jnp.dot(s[:, :D], w2c_ref[...], preferred_element_type=jnp.float32)
          + jnp.dot(s[:, D:D + 16], b16_ref[...],
                    preferred_element_type=jnp.float32)
          + ub_ref[...])
    h2_ref[...] = h2
    pq_ref[...] = jnp.dot(h2, w1ab_ref[...], preferred_element_type=jnp.float32)


def _upd_body_last(h_ref, s_ref, wua_ref, w2c_ref, b16_ref, ub_ref, h2_ref):
    s = s_ref[...]
    h2_ref[...] = (
        jnp.dot(h_ref[...], wua_ref[...], preferred_element_type=jnp.float32)
        + jnp.dot(s[:, :D], w2c_ref[...], preferred_element_type=jnp.float32)
        + jnp.dot(s[:, D:D + 16], b16_ref[...],
                  preferred_element_type=jnp.float32)
        + ub_ref[...])


def _update(h, S, Wua, W2c, B16, ub, W1ab, last):
    grid = 8
    blk = N_NODES // grid
    win = [
        pl.BlockSpec((blk, D), lambda i: (i, 0)),
        pl.BlockSpec((blk, D2), lambda i: (i, 0)),
        pl.BlockSpec((D, D), lambda i: (0, 0)),
        pl.BlockSpec((D, D), lambda i: (0, 0)),
        pl.BlockSpec((16, D), lambda i: (0, 0)),
        pl.BlockSpec((1, D), lambda i: (0, 0)),
    ]
    if last:
        return pl.pallas_call(
            _upd_body_last,
            grid=(grid,),
            in_specs=win,
            out_specs=pl.BlockSpec((blk, D), lambda i: (i, 0)),
            out_shape=jax.ShapeDtypeStruct((N_NODES, D), jnp.float32),
        )(h, S, Wua, W2c, B16, ub.reshape(1, -1))
    return pl.pallas_call(
        _upd_body_full,
        grid=(grid,),
        in_specs=win + [pl.BlockSpec((D, D2), lambda i: (0, 0))],
        out_specs=[
            pl.BlockSpec((blk, D), lambda i: (i, 0)),
            pl.BlockSpec((blk, D2), lambda i: (i, 0)),
        ],
        out_shape=[
            jax.ShapeDtypeStruct((N_NODES, D), jnp.float32),
            jax.ShapeDtypeStruct((N_NODES, D2), jnp.float32),
        ],
    )(h, S, Wua, W2c, B16, ub.reshape(1, -1), W1ab)


def _lse(x, axis):
    m = jnp.max(x, axis=axis, keepdims=True)
    return m + jnp.log(jnp.sum(jnp.exp(x - m), axis=axis, keepdims=True))


def _sink_body(h_ref, f1_ref, b1_ref, f2_ref, b2_ref, out_ref):
    B = 64
    hb = h_ref[...]
    t = jnp.dot(jax.nn.relu(
        jnp.dot(hb, f1_ref[...], preferred_element_type=jnp.float32)
        + b1_ref[...]), f2_ref[...],
        preferred_element_type=jnp.float32) + b2_ref[...]
    t3 = t.reshape(B, 2 * NPG, 16)
    z16 = jnp.zeros((B, MAX_SET - NPG, 16), jnp.float32)
    mq = jnp.concatenate([t3[:, :NPG, :], z16], axis=1)
    mc = jnp.concatenate([t3[:, NPG:, :], z16], axis=1)
    la = lax.dot_general(mq, mc, (((2,), (2,)), ((0,), (0,))),
                         preferred_element_type=jnp.float32) / TEMP
    for _ in range(SINK_ITERS):
        la = la - _lse(la, 2)
        la = la - _lse(la, 1)
    plan = jnp.exp(la)
    h3 = hb.reshape(B, 2 * NPG, D)
    z64 = jnp.zeros((B, MAX_SET - NPG, D), jnp.float32)
    qf = jnp.concatenate([h3[:, :NPG, :], z64], axis=1)
    cf = jnp.concatenate([h3[:, NPG:, :], z64], axis=1)
    pc = lax.dot_general(plan, cf, (((2,), (1,)), ((0,), (0,))),
                         preferred_element_type=jnp.float32)
    s = -jnp.sum(jax.nn.relu(qf - pc), axis=(1, 2))
    out_ref[...] = s.reshape(1, 1, B)


def _sinkhorn(h, f1, b1, f2, b2):
    grid = 32
    B = 64
    return pl.pallas_call(
        _sink_body,
        grid=(grid,),
        in_specs=[
            pl.BlockSpec((B * 30, D), lambda i: (i, 0)),
            pl.BlockSpec((D, 16), lambda i: (0, 0)),
            pl.BlockSpec((1, 16), lambda i: (0, 0)),
            pl.BlockSpec((16, 16), lambda i: (0, 0)),
            pl.BlockSpec((1, 16), lambda i: (0, 0)),
        ],
        out_specs=pl.BlockSpec((1, 1, B), lambda i: (i, 0, 0)),
        out_shape=jax.ShapeDtypeStruct((grid, 1, B), jnp.float32),
    )(h, f1, b1.reshape(1, -1), f2, b2.reshape(1, -1))


def kernel(node_features, edge_features, from_idx, to_idx, enc_node_W,
           enc_node_b, enc_edge_W, enc_edge_b, msg_W1, msg_b1, msg_W2, msg_b2,
           upd_W, upd_b, fc1_W, fc1_b, fc2_W, fc2_b):
    W1ab = jnp.concatenate([msg_W1[:D], msg_W1[D:2 * D]], axis=1)
    W1c = msg_W1[2 * D:]
    Wec = enc_edge_W @ W1c
    bec = enc_edge_b @ W1c + msg_b1
    Wua = upd_W[:D]
    Wub = upd_W[D:]
    W2c = msg_W2 @ Wub
    B16 = jnp.zeros((16, D), jnp.float32).at[0].set(msg_b2 @ Wub)

    fr = from_idx.astype(jnp.int32).reshape(IDX_ROWS, 120)
    fr = jnp.concatenate(
        [fr, jnp.zeros((IDX_ROWS, 8), jnp.int32)], axis=1)
    tr = to_idx.astype(jnp.int32).reshape(IDX_ROWS, 120)
    r = jnp.arange(IDX_ROWS, dtype=jnp.int32)
    wtile = r // 64
    ppair = (r % 64) // 4
    tpad = (16 * 120 - 120 * (wtile // 2) + 2 * CN * (wtile * 16 + ppair))[:, None]
    tr = jnp.concatenate(
        [tr, jnp.broadcast_to(tpad, (IDX_ROWS, 8))], axis=1)

    h, PQ, E1p = _encoder(node_features, edge_features, enc_node_W,
                          enc_node_b, W1ab, Wec, bec)
    for l in range(N_LAYERS):
        S = _edge_stage(PQ, E1p, fr, tr)
        if l < N_LAYERS - 1:
            h, PQ = _update(h, S, Wua, W2c, B16, upd_b, W1ab, last=False)
        else:
            h = _update(h, S, Wua, W2c, B16, upd_b, W1ab, last=True)

    out = _sinkhorn(h, fc1_W, fc1_b, fc2_W, fc2_b)
    return out.reshape(N_PAIRS)


# speedup vs baseline: 6.1032x; 2.0125x over previous
import functools

import jax
import jax.numpy as jnp
from jax import lax
from jax.experimental import pallas as pl
from jax.experimental.pallas import tpu as pltpu
from jax.experimental.pallas import tpu_sc as plsc

N_PAIRS = 2048
N_GRAPHS = 2 * N_PAIRS
NPG = 15
MAX_SET = 20
N_NODES = N_GRAPHS * NPG
EPG = 60
N_EDGES = N_GRAPHS * EPG
D = 64
D2 = 128
N_LAYERS = 5
TEMP = 0.1
SINK_ITERS = 20

NW = 32
G_CHUNK = 4
CHUNKS = N_GRAPHS // (NW * G_CHUNK)
CN = G_CHUNK * NPG
CE = G_CHUNK * EPG
CEP = 256
JROWS = CEP // 128
IDX_ROWS = N_EDGES // 120


def _enc_body(nf_ref, ef_ref, wn_ref, bn_ref, w1ab_ref, wec_ref,
              bec_ref, h_ref, pq_ref, e1_ref):
    h = jnp.dot(nf_ref[...], wn_ref[...],
                preferred_element_type=jnp.float32) + bn_ref[...]
    h_ref[...] = h
    pq_ref[...] = jnp.dot(h, w1ab_ref[...], preferred_element_type=jnp.float32)
    e1 = jnp.dot(ef_ref[...], wec_ref[...],
                 preferred_element_type=jnp.float32) + bec_ref[...]
    e3 = e1.reshape(64, 120, D)
    e3 = jnp.concatenate([e3, jnp.zeros((64, 8, D), jnp.float32)], axis=1)
    e1_ref[...] = e3.reshape(64 * 128, D)


def _encoder(nf, ef, Wn, bn, W1ab, Wec, bec):
    grid = 32
    nblk = N_NODES // grid
    eblk = N_EDGES // grid
    return pl.pallas_call(
        _enc_body,
        grid=(grid,),
        in_specs=[
            pl.BlockSpec((nblk, 32), lambda i: (i, 0)),
            pl.BlockSpec((eblk, 4), lambda i: (i, 0)),
            pl.BlockSpec((32, D), lambda i: (0, 0)),
            pl.BlockSpec((1, D), lambda i: (0, 0)),
            pl.BlockSpec((D, D2), lambda i: (0, 0)),
            pl.BlockSpec((4, D), lambda i: (0, 0)),
            pl.BlockSpec((1, D), lambda i: (0, 0)),
        ],
        out_specs=[
            pl.BlockSpec((nblk, D), lambda i: (i, 0)),
            pl.BlockSpec((nblk, D2), lambda i: (i, 0)),
            pl.BlockSpec((64 * 128, D), lambda i: (i, 0)),
        ],
        out_shape=[
            jax.ShapeDtypeStruct((N_NODES, D), jnp.float32),
            jax.ShapeDtypeStruct((N_NODES, D2), jnp.float32),
            jax.ShapeDtypeStruct((IDX_ROWS * 128, D), jnp.float32),
        ],
    )(nf, ef, Wn, bn.reshape(1, -1), W1ab, Wec, bec.reshape(1, -1))


def _edge_body(pq_hbm, e1_hbm, fidx_hbm, tidx_hbm, s_hbm,
               pq_v, e1_v, s_v, fidx_s, tidx_s, fidx_sh, tidx_sh):
    sid = lax.axis_index("s")
    w = sid * 2 + lax.axis_index("c")
    zero16 = jnp.zeros((16,), jnp.float32)
    one16 = jnp.ones((16,), jnp.float32)

    pltpu.sync_copy(fidx_hbm.at[pl.ds(w * 64, 64)],
                    fidx_sh.at[pl.ds(sid * 64, 64)])
    pltpu.sync_copy(tidx_hbm.at[pl.ds(w * 64, 64)],
                    tidx_sh.at[pl.ds(sid * 64, 64)])

    def chunk_body(cc, _):
        node0 = 120 * (w * 16 + cc)
        e1row0 = 512 * (w * 16 + cc)
        srow = sid * 64 + 4 * cc
        pltpu.sync_copy(pq_hbm.at[pl.ds(node0, 120)], pq_v)
        pltpu.sync_copy(e1_hbm.at[pl.ds(e1row0, 512)], e1_v)
        pltpu.sync_copy(fidx_sh.at[pl.ds(srow, 4)], fidx_s)
        pltpu.sync_copy(tidx_sh.at[pl.ds(srow, 4)], tidx_s)

        def zero_body(i, _):
            for c5 in range(5):
                s_v[i, pl.ds(c5 * 16, 16)] = zero16
            return 0
        lax.fori_loop(0, 120, zero_body, 0, unroll=4)

        for sub in range(4):
            def edge_body(j, _):
                f = fidx_s[sub, j] - node0
                t = tidx_s[sub, j] - node0
                er = 128 * sub + j
                for c4 in range(4):
                    sl = pl.ds(c4 * 16, 16)
                    v = (pq_v[f, sl] + pq_v[t, pl.ds(64 + c4 * 16, 16)]
                         + e1_v[er, sl])
                    s_v[t, sl] = s_v[t, sl] + jnp.maximum(v, 0.0)
                s_v[t, pl.ds(64, 16)] = s_v[t, pl.ds(64, 16)] + one16
                return 0
            lax.fori_loop(0, 120, edge_body, 0)

        pltpu.sync_copy(s_v, s_hbm.at[pl.ds(node0, 120)])
        return 0

    lax.fori_loop(0, 16, chunk_body, 0)


def _edge_stage(PQ, E1p, fr, tr):
    mesh = plsc.VectorSubcoreMesh(core_axis_name="c", subcore_axis_name="s")
    f = functools.partial(
        pl.kernel,
        out_type=jax.ShapeDtypeStruct((N_NODES, 80), jnp.float32),
        mesh=mesh,
        scratch_types=[
            pltpu.VMEM((120, D2), jnp.float32),
            pltpu.VMEM((512, D), jnp.float32),
            pltpu.VMEM((120, 80), jnp.float32),
            pltpu.SMEM((4, 128), jnp.int32),
            pltpu.SMEM((4, 128), jnp.int32),
            pltpu.VMEM_SHARED((16 * 64, 128), jnp.int32),
            pltpu.VMEM_SHARED((16 * 64, 128), jnp.int32),
        ],
    )(_edge_body)
    return f(PQ, E1p, fr, tr)


def _upd_body_full(h_ref, s_ref, wua_ref, w2c_ref, b16_ref, ub_ref,
                   w1ab_ref, h2_ref, pq_ref):
    s = s_ref[...]
    h2 = (jnp.dot(h_ref[...], wua_ref[...], preferred_element_type=jnp.float32)
          + jnp.dot(s[:, :D], w2c_ref[...], preferred_element_type=jnp.float32)
          + jnp.dot(s[:, D:D + 16], b16_ref[...],
                    preferred_element_type=jnp.float32)
          + ub_ref[...])
    h2_ref[...] = h2
    pq_ref[...] = jnp.dot(h2, w1ab_ref[...], preferred_element_type=jnp.float32)


def _upd_body_last(h_ref, s_ref, wua_ref, w2c_ref, b16_ref, ub_ref, h2_ref):
    s = s_ref[...]
    h2_ref[...] = (
        jnp.dot(h_ref[...], wua_ref[...], preferred_element_type=jnp.float32)
        + jnp.dot(s[:, :D], w2c_ref[...], preferred_element_type=jnp.float32)
        + jnp.dot(s[:, D:D + 16], b16_ref[...],
                  preferred_element_type=jnp.float32)
        + ub_ref[...])


def _update(h, S, Wua, W2c, B16, ub, W1ab, last):
    grid = 8
    blk = N_NODES // grid
    win = [
        pl.BlockSpec((blk, D), lambda i: (i, 0)),
        pl.BlockSpec((blk, 80), lambda i: (i, 0)),
        pl.BlockSpec((D, D), lambda i: (0, 0)),
        pl.BlockSpec((D, D), lambda i: (0, 0)),
        pl.BlockSpec((16, D), lambda i: (0, 0)),
        pl.BlockSpec((1, D), lambda i: (0, 0)),
    ]
    if last:
        return pl.pallas_call(
            _upd_body_last,
            grid=(grid,),
            in_specs=win,
            out_specs=pl.BlockSpec((blk, D), lambda i: (i, 0)),
            out_shape=jax.ShapeDtypeStruct((N_NODES, D), jnp.float32),
        )(h, S, Wua, W2c, B16, ub.reshape(1, -1))
    return pl.pallas_call(
        _upd_body_full,
        grid=(grid,),
        in_specs=win + [pl.BlockSpec((D, D2), lambda i: (0, 0))],
        out_specs=[
            pl.BlockSpec((blk, D), lambda i: (i, 0)),
            pl.BlockSpec((blk, D2), lambda i: (i, 0)),
        ],
        out_shape=[
            jax.ShapeDtypeStruct((N_NODES, D), jnp.float32),
            jax.ShapeDtypeStruct((N_NODES, D2), jnp.float32),
        ],
    )(h, S, Wua, W2c, B16, ub.reshape(1, -1), W1ab)


def _lse(x, axis):
    m = jnp.max(x, axis=axis, keepdims=True)
    return m + jnp.log(jnp.sum(jnp.exp(x - m), axis=axis, keepdims=True))


def _sink_body(h_ref, f1_ref, b1_ref, f2_ref, b2_ref, out_ref):
    B = 64
    hb = h_ref[...]
    t = jnp.dot(jax.nn.relu(
        jnp.dot(hb, f1_ref[...], preferred_element_type=jnp.float32)
        + b1_ref[...]), f2_ref[...],
        preferred_element_type=jnp.float32) + b2_ref[...]
    t3 = t.reshape(B, 2 * NPG, 16)
    z16 = jnp.zeros((B, MAX_SET - NPG, 16), jnp.float32)
    mq = jnp.concatenate([t3[:, :NPG, :], z16], axis=1)
    mc = jnp.concatenate([t3[:, NPG:, :], z16], axis=1)
    la = lax.dot_general(mq, mc, (((2,), (2,)), ((0,), (0,))),
                         preferred_element_type=jnp.float32) / TEMP
    for _ in range(SINK_ITERS):
        la = la - _lse(la, 2)
        la = la - _lse(la, 1)
    plan = jnp.exp(la)
    h3 = hb.reshape(B, 2 * NPG, D)
    z64 = jnp.zeros((B, MAX_SET - NPG, D), jnp.float32)
    qf = jnp.concatenate([h3[:, :NPG, :], z64], axis=1)
    cf = jnp.concatenate([h3[:, NPG:, :], z64], axis=1)
    pc = lax.dot_general(plan, cf, (((2,), (1,)), ((0,), (0,))),
                         preferred_element_type=jnp.float32)
    s = -jnp.sum(jax.nn.relu(qf - pc), axis=(1, 2))
    out_ref[...] = s.reshape(1, 1, B)


def _sinkhorn(h, f1, b1, f2, b2):
    grid = 32
    B = 64
    return pl.pallas_call(
        _sink_body,
        grid=(grid,),
        in_specs=[
            pl.BlockSpec((B * 30, D), lambda i: (i, 0)),
            pl.BlockSpec((D, 16), lambda i: (0, 0)),
            pl.BlockSpec((1, 16), lambda i: (0, 0)),
            pl.BlockSpec((16, 16), lambda i: (0, 0)),
            pl.BlockSpec((1, 16), lambda i: (0, 0)),
        ],
        out_specs=pl.BlockSpec((1, 1, B), lambda i: (i, 0, 0)),
        out_shape=jax.ShapeDtypeStruct((grid, 1, B), jnp.float32),
    )(h, f1, b1.reshape(1, -1), f2, b2.reshape(1, -1))


def kernel(node_features, edge_features, from_idx, to_idx, enc_node_W,
           enc_node_b, enc_edge_W, enc_edge_b, msg_W1, msg_b1, msg_W2, msg_b2,
           upd_W, upd_b, fc1_W, fc1_b, fc2_W, fc2_b):
    W1ab = jnp.concatenate([msg_W1[:D], msg_W1[D:2 * D]], axis=1)
    W1c = msg_W1[2 * D:]
    Wec = enc_edge_W @ W1c
    bec = enc_edge_b @ W1c + msg_b1
    Wua = upd_W[:D]
    Wub = upd_W[D:]
    W2c = msg_W2 @ Wub
    B16 = jnp.zeros((16, D), jnp.float32).at[0].set(msg_b2 @ Wub)

    fr = from_idx.astype(jnp.int32).reshape(IDX_ROWS, 120)
    fr = jnp.concatenate(
        [fr, jnp.zeros((IDX_ROWS, 8), jnp.int32)], axis=1)
    tr = to_idx.astype(jnp.int32).reshape(IDX_ROWS, 120)
    tr = jnp.concatenate(
        [tr, jnp.zeros((IDX_ROWS, 8), jnp.int32)], axis=1)

    h, PQ, E1p = _encoder(node_features, edge_features, enc_node_W,
                          enc_node_b, W1ab, Wec, bec)
    for l in range(N_LAYERS):
        S = _edge_stage(PQ, E1p, fr, tr)
        if l < N_LAYERS - 1:
            h, PQ = _update(h, S, Wua, W2c, B16, upd_b, W1ab, last=False)
        else:
            h = _update(h, S, Wua, W2c, B16, upd_b, W1ab, last=True)

    out = _sinkhorn(h, fc1_W, fc1_b, fc2_W, fc2_b)
    return out.reshape(N_PAIRS)
